# Initial kernel scaffold; baseline (speedup 1.0000x reference)
import jax
import jax.numpy as jnp
from jax.experimental import pallas as pl


def kernel(query, q_weights, k_weights, v_weights, q_bias, k_bias, v_bias):
    raise NotImplementedError("write your pallas kernel here")


# speedup vs baseline: 55.8645x; 55.8645x over previous
import functools
import math

import jax
import jax.numpy as jnp
import numpy as np
from jax.experimental import pallas as pl

NLAT = 61
NLON = 120
C = 256
NPIX = NLAT * NLON
WIN = 3 * NLON


@functools.lru_cache(maxsize=1)
def _log_mask() -> np.ndarray:
    theta = np.linspace(0.0, np.pi, NLAT)
    dtheta = np.pi / (NLAT - 1)
    w = np.sin(theta) * dtheta
    w[0] *= 0.5
    w[-1] *= 0.5
    w = np.maximum(w, 1e-4)
    quad_w = (2.0 * np.pi * w / NLON).astype(np.float64)
    log_qw = np.log(quad_w)

    phi = np.linspace(0.0, 2.0 * np.pi, NLON, endpoint=False)
    cut = (math.pi / (NLAT - 1)) * (1.0 + 1e-5)
    ct = np.cos(theta)[:, None]
    st = np.sin(theta)[:, None]
    cp = np.cos(phi)[None, :]

    L = np.full((NLAT, NLON, WIN), -1e30, dtype=np.float32)
    wo = np.arange(NLON)
    for ho in range(NLAT):
        cosd = math.cos(theta[ho]) * ct + math.sin(theta[ho]) * st * cp
        dist = np.arccos(np.clip(cosd, -1.0, 1.0))
        hi, wi = np.nonzero(dist <= cut)
        base = min(max(ho - 1, 0), NLAT - 3)
        for h, wv in zip(hi, wi):
            r = h - base
            lon = (wv + wo) % NLON
            L[ho, wo, r * NLON + lon] = log_qw[h]
    return L


def _proj_kernel(x_ref, w_ref, b_ref, o_ref):
    o_ref[...] = (
        jnp.dot(x_ref[...], w_ref[...], preferred_element_type=jnp.float32)
        + b_ref[...]
    )


def _attn_kernel(qkv_k_ref, qkv_v_ref, q_ref, l_ref, o_ref):
    ho = pl.program_id(0)
    base = jnp.clip(ho - 1, 0, NLAT - 3) * NLON
    k_win = qkv_k_ref[pl.ds(base, WIN), :]
    v_win = qkv_v_ref[pl.ds(base, WIN), :]
    q = q_ref[...]
    corr = jax.lax.dot_general(
        q, k_win, (((1,), (1,)), ((), ())), preferred_element_type=jnp.float32
    )
    s = corr + l_ref[0]
    m = jnp.max(s, axis=1, keepdims=True)
    a = jnp.exp(s - m)
    attn = a / jnp.sum(a, axis=1, keepdims=True)
    o_ref[...] = jax.lax.dot_general(
        attn, v_win, (((1,), (0,)), ((), ())), preferred_element_type=jnp.float32
    )


def kernel(query, q_weights, k_weights, v_weights, q_bias, k_bias, v_bias):
    scale = math.sqrt(1.0 / C)
    x = query[0].reshape(C, NPIX).T
    w_cat = jnp.concatenate(
        [(scale * q_weights).T, k_weights.T, v_weights.T], axis=1
    )
    b_cat = jnp.concatenate([q_bias, k_bias, v_bias]).reshape(1, 3 * C)

    m_t = NPIX // 5
    qkv = pl.pallas_call(
        _proj_kernel,
        grid=(NPIX // m_t,),
        in_specs=[
            pl.BlockSpec((m_t, C), lambda i: (i, 0)),
            pl.BlockSpec((C, 3 * C), lambda i: (0, 0)),
            pl.BlockSpec((1, 3 * C), lambda i: (0, 0)),
        ],
        out_specs=pl.BlockSpec((m_t, 3 * C), lambda i: (i, 0)),
        out_shape=jax.ShapeDtypeStruct((NPIX, 3 * C), jnp.float32),
    )(x, w_cat, b_cat)

    l_mask = jnp.asarray(_log_mask())
    out = pl.pallas_call(
        _attn_kernel,
        grid=(NLAT,),
        in_specs=[
            pl.BlockSpec((NPIX, C), lambda h: (0, 1)),
            pl.BlockSpec((NPIX, C), lambda h: (0, 2)),
            pl.BlockSpec((NLON, C), lambda h: (h, 0)),
            pl.BlockSpec((1, NLON, WIN), lambda h: (h, 0, 0)),
        ],
        out_specs=pl.BlockSpec((NLON, C), lambda h: (h, 0)),
        out_shape=jax.ShapeDtypeStruct((NPIX, C), jnp.float32),
    )(qkv, qkv, qkv, l_mask)

    return out.T.reshape(1, C, NLAT, NLON)


# speedup vs baseline: 57.6942x; 1.0328x over previous
import functools
import math

import jax
import jax.numpy as jnp
import numpy as np
from jax.experimental import pallas as pl
from jax.experimental.pallas import tpu as pltpu

NLAT = 61
NLON = 120
C = 256
NPIX = NLAT * NLON
WIN = 3 * NLON
NEG = -1e30


@functools.lru_cache(maxsize=1)
def _mask_tables():
    theta = np.linspace(0.0, np.pi, NLAT)
    dtheta = np.pi / (NLAT - 1)
    w = np.sin(theta) * dtheta
    w[0] *= 0.5
    w[-1] *= 0.5
    w = np.maximum(w, 1e-4)
    quad_w = (2.0 * np.pi * w / NLON).astype(np.float64)
    log_qw = np.log(quad_w)

    phi = np.linspace(0.0, 2.0 * np.pi, NLON, endpoint=False)
    cut = (math.pi / (NLAT - 1)) * (1.0 + 1e-5)
    ct = np.cos(theta)[:, None]
    st = np.sin(theta)[:, None]
    cp = np.cos(phi)[None, :]

    lon = np.arange(NLON)
    cd = np.minimum(lon, NLON - lon)

    tbl = np.zeros((NLAT, 8), dtype=np.float32)
    for ho in range(NLAT):
        cosd = math.cos(theta[ho]) * ct + math.sin(theta[ho]) * st * cp
        dist = np.arccos(np.clip(cosd, -1.0, 1.0))
        hi, wi = np.nonzero(dist <= cut)
        base = min(max(ho - 1, 0), NLAT - 3)
        for r in range(3):
            sel = wi[hi == base + r]
            if len(sel) == 0:
                tbl[ho, r] = -1.0
                tbl[ho, 3 + r] = 0.0
                continue
            d = int(cd[sel].max())
            assert set(sel.tolist()) == {l for l in range(NLON) if cd[l] <= d}
            tbl[ho, r] = float(d)
            tbl[ho, 3 + r] = float(log_qw[base + r])

    wo = np.arange(NLON)[:, None]
    l_abs = np.arange(WIN)[None, :] % NLON
    delta = (l_abs - wo) % NLON
    dmap = np.minimum(delta, NLON - delta).astype(np.float32)
    return dmap, tbl


def _proj_kernel(x_ref, w_ref, b_ref, o_ref):
    o_ref[...] = (
        jnp.dot(x_ref[...], w_ref[...], preferred_element_type=jnp.float32)
        + b_ref[...]
    )


def _attn_kernel(tbl_ref, qkv_k_ref, qkv_v_ref, q_ref, dmap_ref, o_ref):
    ho = pl.program_id(0)
    base = jnp.clip(ho - 1, 0, NLAT - 3) * NLON
    k_win = qkv_k_ref[pl.ds(base, WIN), :]
    v_win = qkv_v_ref[pl.ds(base, WIN), :]
    corr = jax.lax.dot_general(
        q_ref[...], k_win, (((1,), (1,)), ((), ())),
        preferred_element_type=jnp.float32,
    )

    col = jax.lax.broadcasted_iota(jnp.int32, (1, WIN), 1)
    def band(v0, v1, v2):
        return jnp.where(col < NLON, v0, jnp.where(col < 2 * NLON, v1, v2))
    dvec = band(tbl_ref[ho, 0], tbl_ref[ho, 1], tbl_ref[ho, 2])
    lqvec = band(tbl_ref[ho, 3], tbl_ref[ho, 4], tbl_ref[ho, 5])

    s = jnp.where(dmap_ref[...] <= dvec, corr + lqvec, NEG)
    m = jnp.max(s, axis=1, keepdims=True)
    a = jnp.exp(s - m)
    attn = a / jnp.sum(a, axis=1, keepdims=True)
    o_ref[...] = jax.lax.dot_general(
        attn, v_win, (((1,), (0,)), ((), ())), preferred_element_type=jnp.float32
    )


def kernel(query, q_weights, k_weights, v_weights, q_bias, k_bias, v_bias):
    scale = math.sqrt(1.0 / C)
    x = query[0].reshape(C, NPIX).T
    w_cat = jnp.concatenate(
        [(scale * q_weights).T, k_weights.T, v_weights.T], axis=1
    )
    b_cat = jnp.concatenate([q_bias, k_bias, v_bias]).reshape(1, 3 * C)

    m_t = NPIX // 5
    qkv = pl.pallas_call(
        _proj_kernel,
        grid=(NPIX // m_t,),
        in_specs=[
            pl.BlockSpec((m_t, C), lambda i: (i, 0)),
            pl.BlockSpec((C, 3 * C), lambda i: (0, 0)),
            pl.BlockSpec((1, 3 * C), lambda i: (0, 0)),
        ],
        out_specs=pl.BlockSpec((m_t, 3 * C), lambda i: (i, 0)),
        out_shape=jax.ShapeDtypeStruct((NPIX, 3 * C), jnp.float32),
    )(x, w_cat, b_cat)

    dmap_np, tbl_np = _mask_tables()
    out = pl.pallas_call(
        _attn_kernel,
        grid=(NLAT,),
        in_specs=[
            pl.BlockSpec(memory_space=pltpu.SMEM),
            pl.BlockSpec((NPIX, C), lambda h: (0, 1)),
            pl.BlockSpec((NPIX, C), lambda h: (0, 2)),
            pl.BlockSpec((NLON, C), lambda h: (h, 0)),
            pl.BlockSpec((NLON, WIN), lambda h: (0, 0)),
        ],
        out_specs=pl.BlockSpec((NLON, C), lambda h: (h, 0)),
        out_shape=jax.ShapeDtypeStruct((NPIX, C), jnp.float32),
    )(jnp.asarray(tbl_np), qkv, qkv, qkv, jnp.asarray(dmap_np))

    return out.T.reshape(1, C, NLAT, NLON)


# speedup vs baseline: 63.6034x; 1.1024x over previous
import functools
import math

import jax
import jax.numpy as jnp
import numpy as np
from jax.experimental import pallas as pl
from jax.experimental.pallas import tpu as pltpu

NLAT = 61
NLON = 120
C = 256
NPIX = NLAT * NLON
WIN = 3 * NLON
NEG = -1e30
NLAT_P = 64
NPIX_P = NLAT_P * NLON
RPS = 16


@functools.lru_cache(maxsize=1)
def _mask_tables():
    theta = np.linspace(0.0, np.pi, NLAT)
    dtheta = np.pi / (NLAT - 1)
    w = np.sin(theta) * dtheta
    w[0] *= 0.5
    w[-1] *= 0.5
    w = np.maximum(w, 1e-4)
    quad_w = (2.0 * np.pi * w / NLON).astype(np.float64)
    log_qw = np.log(quad_w)

    phi = np.linspace(0.0, 2.0 * np.pi, NLON, endpoint=False)
    cut = (math.pi / (NLAT - 1)) * (1.0 + 1e-5)
    ct = np.cos(theta)[:, None]
    st = np.sin(theta)[:, None]
    cp = np.cos(phi)[None, :]

    lon = np.arange(NLON)
    cd = np.minimum(lon, NLON - lon)

    tbl = np.zeros((NLAT, 8), dtype=np.float32)
    for ho in range(NLAT):
        cosd = math.cos(theta[ho]) * ct + math.sin(theta[ho]) * st * cp
        dist = np.arccos(np.clip(cosd, -1.0, 1.0))
        hi, wi = np.nonzero(dist <= cut)
        base = min(max(ho - 1, 0), NLAT - 3)
        for r in range(3):
            sel = wi[hi == base + r]
            if len(sel) == 0:
                tbl[ho, r] = -1.0
                tbl[ho, 3 + r] = 0.0
                continue
            d = int(cd[sel].max())
            assert set(sel.tolist()) == {l for l in range(NLON) if cd[l] <= d}
            tbl[ho, r] = float(d)
            tbl[ho, 3 + r] = float(log_qw[base + r])

    wo = np.arange(NLON)[:, None]
    l_abs = np.arange(WIN)[None, :] % NLON
    delta = (l_abs - wo) % NLON
    dmap = np.minimum(delta, NLON - delta).astype(np.float32)
    return dmap, tbl


def _proj_kernel(x_ref, w_ref, b_ref, o_ref):
    o_ref[...] = (
        jnp.dot(x_ref[...], w_ref[...], preferred_element_type=jnp.float32)
        + b_ref[...]
    )


def _attn_kernel(tbl_ref, qkv_k_ref, qkv_v_ref, q_ref, dmap_ref, o_ref):
    g = pl.program_id(0)
    col = jax.lax.broadcasted_iota(jnp.int32, (1, WIN), 1)
    dmap = dmap_ref[...]

    def band(v0, v1, v2):
        return jnp.where(col < NLON, v0, jnp.where(col < 2 * NLON, v1, v2))

    for r in range(RPS):
        ho = g * RPS + r
        hoc = jnp.minimum(ho, NLAT - 1)
        base = jnp.clip(ho - 1, 0, NLAT - 3) * NLON
        k_win = qkv_k_ref[pl.ds(base, WIN), :]
        v_win = qkv_v_ref[pl.ds(base, WIN), :]
        q = q_ref[r * NLON:(r + 1) * NLON, :]
        corr = jax.lax.dot_general(
            q, k_win, (((1,), (1,)), ((), ())),
            preferred_element_type=jnp.float32,
        )
        dvec = band(tbl_ref[hoc, 0], tbl_ref[hoc, 1], tbl_ref[hoc, 2])
        lqvec = band(tbl_ref[hoc, 3], tbl_ref[hoc, 4], tbl_ref[hoc, 5])
        s = jnp.where(dmap <= dvec, corr + lqvec, NEG)
        m = jnp.max(s, axis=1, keepdims=True)
        a = jnp.exp(s - m)
        attn = a / jnp.sum(a, axis=1, keepdims=True)
        o_ref[r * NLON:(r + 1) * NLON, :] = jax.lax.dot_general(
            attn, v_win, (((1,), (0,)), ((), ())),
            preferred_element_type=jnp.float32,
        )


def kernel(query, q_weights, k_weights, v_weights, q_bias, k_bias, v_bias):
    scale = math.sqrt(1.0 / C)
    x = jnp.pad(query[0].reshape(C, NPIX), ((0, 0), (0, NPIX_P - NPIX))).T
    w_cat = jnp.concatenate(
        [(scale * q_weights).T, k_weights.T, v_weights.T], axis=1
    )
    b_cat = jnp.concatenate([q_bias, k_bias, v_bias]).reshape(1, 3 * C)

    m_t = NPIX_P // 5
    qkv = pl.pallas_call(
        _proj_kernel,
        grid=(NPIX_P // m_t,),
        in_specs=[
            pl.BlockSpec((m_t, C), lambda i: (i, 0)),
            pl.BlockSpec((C, 3 * C), lambda i: (0, 0)),
            pl.BlockSpec((1, 3 * C), lambda i: (0, 0)),
        ],
        out_specs=pl.BlockSpec((m_t, 3 * C), lambda i: (i, 0)),
        out_shape=jax.ShapeDtypeStruct((NPIX_P, 3 * C), jnp.float32),
    )(x, w_cat, b_cat)

    dmap_np, tbl_np = _mask_tables()
    out = pl.pallas_call(
        _attn_kernel,
        grid=(NLAT_P // RPS,),
        in_specs=[
            pl.BlockSpec(memory_space=pltpu.SMEM),
            pl.BlockSpec((NPIX_P, C), lambda h: (0, 1)),
            pl.BlockSpec((NPIX_P, C), lambda h: (0, 2)),
            pl.BlockSpec((RPS * NLON, C), lambda h: (h, 0)),
            pl.BlockSpec((NLON, WIN), lambda h: (0, 0)),
        ],
        out_specs=pl.BlockSpec((RPS * NLON, C), lambda h: (h, 0)),
        out_shape=jax.ShapeDtypeStruct((NPIX_P, C), jnp.float32),
    )(jnp.asarray(tbl_np), qkv, qkv, qkv, jnp.asarray(dmap_np))

    return out[:NPIX].T.reshape(1, C, NLAT, NLON)


# speedup vs baseline: 69.7994x; 1.0974x over previous
import functools
import math

import jax
import jax.numpy as jnp
import numpy as np
from jax.experimental import pallas as pl
from jax.experimental.pallas import tpu as pltpu

NLAT = 61
NLON = 120
C = 256
NPIX = NLAT * NLON
WIN = 3 * NLON
NEG = -1e30
NLAT_P = 64
NPIX_P = NLAT_P * NLON
RPS = 16


@functools.lru_cache(maxsize=1)
def _mask_tables():
    theta = np.linspace(0.0, np.pi, NLAT)
    dtheta = np.pi / (NLAT - 1)
    w = np.sin(theta) * dtheta
    w[0] *= 0.5
    w[-1] *= 0.5
    w = np.maximum(w, 1e-4)
    quad_w = (2.0 * np.pi * w / NLON).astype(np.float64)
    log_qw = np.log(quad_w)

    phi = np.linspace(0.0, 2.0 * np.pi, NLON, endpoint=False)
    cut = (math.pi / (NLAT - 1)) * (1.0 + 1e-5)
    ct = np.cos(theta)[:, None]
    st = np.sin(theta)[:, None]
    cp = np.cos(phi)[None, :]

    lon = np.arange(NLON)
    cd = np.minimum(lon, NLON - lon)

    tbl = np.zeros((NLAT, 8), dtype=np.float32)
    for ho in range(NLAT):
        cosd = math.cos(theta[ho]) * ct + math.sin(theta[ho]) * st * cp
        dist = np.arccos(np.clip(cosd, -1.0, 1.0))
        hi, wi = np.nonzero(dist <= cut)
        base = min(max(ho - 1, 0), NLAT - 3)
        for r in range(3):
            sel = wi[hi == base + r]
            if len(sel) == 0:
                tbl[ho, r] = -1.0
                tbl[ho, 3 + r] = 0.0
                continue
            d = int(cd[sel].max())
            assert set(sel.tolist()) == {l for l in range(NLON) if cd[l] <= d}
            tbl[ho, r] = float(d)
            tbl[ho, 3 + r] = float(log_qw[base + r])

    wo = np.arange(NLON)[:, None]
    l_abs = np.arange(WIN)[None, :] % NLON
    delta = (l_abs - wo) % NLON
    dmap = np.minimum(delta, NLON - delta).astype(np.float32)
    return dmap, tbl


def _proj_kernel(x_ref, w_ref, b_ref, o_ref):
    o_ref[...] = (
        jnp.dot(x_ref[...], w_ref[...], preferred_element_type=jnp.float32)
        + b_ref[...]
    ).astype(jnp.bfloat16)


def _attn_kernel(tbl_ref, qkv_k_ref, qkv_v_ref, q_ref, dmap_ref, o_ref):
    g = pl.program_id(0)
    col = jax.lax.broadcasted_iota(jnp.int32, (1, WIN), 1)
    dmap = dmap_ref[...]

    def band(v0, v1, v2):
        return jnp.where(col < NLON, v0, jnp.where(col < 2 * NLON, v1, v2))

    for r in range(RPS):
        ho = g * RPS + r
        hoc = jnp.minimum(ho, NLAT - 1)
        base = jnp.clip(ho - 1, 0, NLAT - 3) * NLON
        k_win = qkv_k_ref[pl.ds(base, WIN), :]
        v_win = qkv_v_ref[pl.ds(base, WIN), :]
        q = q_ref[r * NLON:(r + 1) * NLON, :]
        corr = jax.lax.dot_general(
            q, k_win, (((1,), (1,)), ((), ())),
            preferred_element_type=jnp.float32,
        )
        dvec = band(tbl_ref[hoc, 0], tbl_ref[hoc, 1], tbl_ref[hoc, 2])
        lqvec = band(tbl_ref[hoc, 3], tbl_ref[hoc, 4], tbl_ref[hoc, 5])
        s = jnp.where(dmap <= dvec, corr + lqvec, NEG)
        m = jnp.max(s, axis=1, keepdims=True)
        a = jnp.exp(s - m)
        attn = (a / jnp.sum(a, axis=1, keepdims=True)).astype(jnp.bfloat16)
        o_ref[r * NLON:(r + 1) * NLON, :] = jax.lax.dot_general(
            attn, v_win, (((1,), (0,)), ((), ())),
            preferred_element_type=jnp.float32,
        )


def kernel(query, q_weights, k_weights, v_weights, q_bias, k_bias, v_bias):
    scale = math.sqrt(1.0 / C)
    x = jnp.pad(
        query[0].reshape(C, NPIX).astype(jnp.bfloat16),
        ((0, 0), (0, NPIX_P - NPIX)),
    ).T
    w_cat = jnp.concatenate(
        [(scale * q_weights).T, k_weights.T, v_weights.T], axis=1
    ).astype(jnp.bfloat16)
    b_cat = jnp.concatenate([q_bias, k_bias, v_bias]).reshape(1, 3 * C)

    m_t = NPIX_P // 5
    qkv = pl.pallas_call(
        _proj_kernel,
        grid=(NPIX_P // m_t,),
        in_specs=[
            pl.BlockSpec((m_t, C), lambda i: (i, 0)),
            pl.BlockSpec((C, 3 * C), lambda i: (0, 0)),
            pl.BlockSpec((1, 3 * C), lambda i: (0, 0)),
        ],
        out_specs=pl.BlockSpec((m_t, 3 * C), lambda i: (i, 0)),
        out_shape=jax.ShapeDtypeStruct((NPIX_P, 3 * C), jnp.bfloat16),
    )(x, w_cat, b_cat)

    dmap_np, tbl_np = _mask_tables()
    out = pl.pallas_call(
        _attn_kernel,
        grid=(NLAT_P // RPS,),
        in_specs=[
            pl.BlockSpec(memory_space=pltpu.SMEM),
            pl.BlockSpec((NPIX_P, C), lambda h: (0, 1)),
            pl.BlockSpec((NPIX_P, C), lambda h: (0, 2)),
            pl.BlockSpec((RPS * NLON, C), lambda h: (h, 0)),
            pl.BlockSpec((NLON, WIN), lambda h: (0, 0)),
        ],
        out_specs=pl.BlockSpec((RPS * NLON, C), lambda h: (h, 0)),
        out_shape=jax.ShapeDtypeStruct((NPIX_P, C), jnp.float32),
    )(jnp.asarray(tbl_np), qkv, qkv, qkv, jnp.asarray(dmap_np))

    return out[:NPIX].T.reshape(1, C, NLAT, NLON)


# speedup vs baseline: 72.7159x; 1.0418x over previous
import functools
import math

import jax
import jax.numpy as jnp
import numpy as np
from jax.experimental import pallas as pl
from jax.experimental.pallas import tpu as pltpu

NLAT = 61
NLON = 120
C = 256
NPIX = NLAT * NLON
WIN = 3 * NLON
NEG = -1e30
NLAT_P = 64
NPIX_P = NLAT_P * NLON
PSTEPS = 4
ASTEPS = 8
RPS = NLAT_P // ASTEPS


@functools.lru_cache(maxsize=1)
def _mask_tables():
    theta = np.linspace(0.0, np.pi, NLAT)
    dtheta = np.pi / (NLAT - 1)
    w = np.sin(theta) * dtheta
    w[0] *= 0.5
    w[-1] *= 0.5
    w = np.maximum(w, 1e-4)
    quad_w = (2.0 * np.pi * w / NLON).astype(np.float64)
    log_qw = np.log(quad_w)

    phi = np.linspace(0.0, 2.0 * np.pi, NLON, endpoint=False)
    cut = (math.pi / (NLAT - 1)) * (1.0 + 1e-5)
    ct = np.cos(theta)[:, None]
    st = np.sin(theta)[:, None]
    cp = np.cos(phi)[None, :]

    lon = np.arange(NLON)
    cd = np.minimum(lon, NLON - lon)

    tbl = np.zeros((NLAT, 8), dtype=np.float32)
    for ho in range(NLAT):
        cosd = math.cos(theta[ho]) * ct + math.sin(theta[ho]) * st * cp
        dist = np.arccos(np.clip(cosd, -1.0, 1.0))
        hi, wi = np.nonzero(dist <= cut)
        base = min(max(ho - 1, 0), NLAT - 3)
        for r in range(3):
            sel = wi[hi == base + r]
            if len(sel) == 0:
                tbl[ho, r] = -1.0
                tbl[ho, 3 + r] = 0.0
                continue
            d = int(cd[sel].max())
            assert set(sel.tolist()) == {l for l in range(NLON) if cd[l] <= d}
            tbl[ho, r] = float(d)
            tbl[ho, 3 + r] = float(log_qw[base + r])

    wo = np.arange(NLON)[:, None]
    l_abs = np.arange(WIN)[None, :] % NLON
    delta = (l_abs - wo) % NLON
    dmap = np.minimum(delta, NLON - delta).astype(np.float32)
    return dmap, tbl


def _fused_kernel(tbl_ref, x_ref, w_ref, b_ref, dmap_ref, o_ref, qkv_ref):
    step = pl.program_id(0)

    @pl.when(step < PSTEPS)
    def _project():
        for p in range(PSTEPS):

            @pl.when(step == p)
            def _():
                lo = p * 1920
                width = min(1920, NPIX - lo)
                xb = x_ref[:, lo:lo + width].astype(jnp.bfloat16)
                qkv = jax.lax.dot_general(
                    xb, w_ref[...], (((0,), (0,)), ((), ())),
                    preferred_element_type=jnp.float32,
                ) + b_ref[...]
                qkv_ref[lo:lo + width, :] = qkv.astype(jnp.bfloat16)

    @pl.when(step >= PSTEPS)
    def _attend():
        g = step - PSTEPS
        col = jax.lax.broadcasted_iota(jnp.int32, (1, WIN), 1)
        dmap = dmap_ref[...]

        def band(v0, v1, v2):
            return jnp.where(col < NLON, v0, jnp.where(col < 2 * NLON, v1, v2))

        for r in range(RPS):
            ho = g * RPS + r
            hoc = jnp.minimum(ho, NLAT - 1)
            base = jnp.clip(ho - 1, 0, NLAT - 3) * NLON
            k_win = qkv_ref[pl.ds(base, WIN), C:2 * C]
            v_win = qkv_ref[pl.ds(base, WIN), 2 * C:]
            q = qkv_ref[pl.ds(ho * NLON, NLON), :C]
            corr = jax.lax.dot_general(
                q, k_win, (((1,), (1,)), ((), ())),
                preferred_element_type=jnp.float32,
            )
            dvec = band(tbl_ref[hoc, 0], tbl_ref[hoc, 1], tbl_ref[hoc, 2])
            lqvec = band(tbl_ref[hoc, 3], tbl_ref[hoc, 4], tbl_ref[hoc, 5])
            s = jnp.where(dmap <= dvec, corr + lqvec, NEG)
            m = jnp.max(s, axis=1, keepdims=True)
            a = jnp.exp(s - m)
            attn = (a / jnp.sum(a, axis=1, keepdims=True)).astype(jnp.bfloat16)
            o_ref[r * NLON:(r + 1) * NLON, :] = jax.lax.dot_general(
                attn, v_win, (((1,), (0,)), ((), ())),
                preferred_element_type=jnp.float32,
            )


def kernel(query, q_weights, k_weights, v_weights, q_bias, k_bias, v_bias):
    scale = math.sqrt(1.0 / C)
    x = query[0].reshape(C, NPIX)
    w_cat = jnp.concatenate(
        [(scale * q_weights).T, k_weights.T, v_weights.T], axis=1
    ).astype(jnp.bfloat16)
    b_cat = jnp.concatenate([q_bias, k_bias, v_bias]).reshape(1, 3 * C)

    dmap_np, tbl_np = _mask_tables()
    out = pl.pallas_call(
        _fused_kernel,
        grid=(PSTEPS + ASTEPS,),
        in_specs=[
            pl.BlockSpec(memory_space=pltpu.SMEM),
            pl.BlockSpec((C, NPIX), lambda h: (0, 0)),
            pl.BlockSpec((C, 3 * C), lambda h: (0, 0)),
            pl.BlockSpec((1, 3 * C), lambda h: (0, 0)),
            pl.BlockSpec((NLON, WIN), lambda h: (0, 0)),
        ],
        out_specs=pl.BlockSpec(
            (RPS * NLON, C), lambda h: (jnp.maximum(h - PSTEPS, 0), 0)
        ),
        out_shape=jax.ShapeDtypeStruct((NPIX_P, C), jnp.float32),
        scratch_shapes=[pltpu.VMEM((NPIX_P, 3 * C), jnp.bfloat16)],
    )(jnp.asarray(tbl_np), x, w_cat, b_cat, jnp.asarray(dmap_np))

    return out[:NPIX].T.reshape(1, C, NLAT, NLON)
